# baseline (speedup 1.0000x reference)
import functools

import jax
import jax.numpy as jnp
from jax import lax
from jax.experimental import pallas as pl
from jax.experimental.pallas import tpu as pltpu
from jax.experimental.pallas import tpu_sc as plsc

N_NODES = 10000
DIM_IN = 128
DIM_H = 64
DIM_O = 6
N_GRAPH = 64
N_EDGE = 320000

NC, NS, LANES = 2, 16, 16
NW = NC * NS
NP = 10240
ROWS_W = NP // NS
CH = 128
NCH_W = 81
NCH_TOT = NW * NCH_W
EP = NCH_TOT * CH
NB = 4
BLK = 640
BLK2 = BLK // 2
GRID = NP // BLK

_mesh = plsc.VectorSubcoreMesh(
    core_axis_name="c", subcore_axis_name="s", num_cores=NC, num_subcores=NS)
_sc_params = pltpu.CompilerParams(use_tc_tiling_on_sc=False)


@functools.partial(
    pl.kernel,
    out_type=jax.ShapeDtypeStruct((NC * NP, LANES), jnp.float32),
    mesh=_mesh,
    scratch_types=[
        pltpu.VMEM((CH, LANES), jnp.float32),
        pltpu.VMEM((CH, LANES), jnp.float32),
        pltpu.VMEM((NCH_W, CH), jnp.int32),
        pltpu.SemaphoreType.DMA,
        pltpu.VMEM_SHARED((NP, LANES), jnp.float32),
    ],
    compiler_params=_sc_params,
)
def _deg_kernel(dst_hbm, out_hbm, zb, ones_v, didx, ssem, cnt_sp):
    c = lax.axis_index("c")
    s = lax.axis_index("s")
    w = c * NS + s

    def fill(i, _):
        zb[i, :] = jnp.zeros((LANES,), jnp.float32)
        ones_v[i, :] = jnp.ones((LANES,), jnp.float32)
        return 0

    lax.fori_loop(0, CH, fill, 0)
    pltpu.sync_copy(dst_hbm.at[pl.ds(w * NCH_W, NCH_W)], didx)
    for k in range(ROWS_W // CH):
        pltpu.sync_copy(zb, cnt_sp.at[pl.ds(s * ROWS_W + k * CH, CH)])
    plsc.subcore_barrier()

    def pair(i, _):
        for k in range(2):
            j = i * 2 + k
            pltpu.async_copy(ones_v, cnt_sp.at[didx.at[j]], ssem, add=True)

            @pl.when(j >= 4)
            def _():
                pltpu.make_async_copy(ones_v, cnt_sp.at[didx.at[j]],
                                      ssem).wait()
        return 0

    lax.fori_loop(0, (NCH_W - 1) // 2, pair, 0)
    pltpu.async_copy(ones_v, cnt_sp.at[didx.at[NCH_W - 1]], ssem, add=True)
    for _ in range(5):
        pltpu.make_async_copy(ones_v, cnt_sp.at[didx.at[0]], ssem).wait()
    plsc.subcore_barrier()
    pltpu.sync_copy(cnt_sp.at[pl.ds(s * ROWS_W, ROWS_W)],
                    out_hbm.at[pl.ds(c * NP + s * ROWS_W, ROWS_W)])


@functools.partial(
    pl.kernel,
    out_type=jax.ShapeDtypeStruct((NC * NP, DIM_H), jnp.float32),
    mesh=_mesh,
    scratch_types=[
        pltpu.VMEM((CH, DIM_H), jnp.float32),
        pltpu.VMEM((NB, CH, DIM_H), jnp.float32),
        pltpu.VMEM((NCH_W, CH), jnp.int32),
        pltpu.VMEM((NCH_W, CH), jnp.int32),
        pltpu.SemaphoreType.DMA,
        pltpu.VMEM_SHARED((NP, DIM_H), jnp.float32),
    ],
    compiler_params=_sc_params,
)
def _agg_kernel(g_hbm, src_hbm, dst_hbm, out_hbm, zb, rows, sidx, didx, gsem,
                acc_sp):
    c = lax.axis_index("c")
    s = lax.axis_index("s")
    w = c * NS + s

    def fill(i, _):
        for k in range(DIM_H // LANES):
            zb[i, pl.ds(k * LANES, LANES)] = jnp.zeros((LANES,), jnp.float32)
        return 0

    lax.fori_loop(0, CH, fill, 0)
    pltpu.sync_copy(src_hbm.at[pl.ds(w * NCH_W, NCH_W)], sidx)
    pltpu.sync_copy(dst_hbm.at[pl.ds(w * NCH_W, NCH_W)], didx)
    for k in range(ROWS_W // CH):
        pltpu.sync_copy(zb, acc_sp.at[pl.ds(s * ROWS_W + k * CH, CH)])
    plsc.subcore_barrier()

    for b in range(NB - 1):
        pltpu.async_copy(g_hbm.at[sidx.at[b]], rows.at[b], gsem)

    def quad(i, _):
        for k in range(NB):
            j = i * NB + k
            pltpu.make_async_copy(g_hbm.at[sidx.at[k]], rows.at[k],
                                  gsem).wait()
            jn = j + NB - 1

            @pl.when(jn < NCH_W)
            def _():
                bn = (k + NB - 1) % NB
                pltpu.async_copy(g_hbm.at[sidx.at[jn]], rows.at[bn], gsem)

            pltpu.sync_copy(rows.at[k], acc_sp.at[didx.at[j]], add=True)
        return 0

    lax.fori_loop(0, (NCH_W - 1) // NB, quad, 0)
    pltpu.make_async_copy(g_hbm.at[sidx.at[0]], rows.at[0], gsem).wait()
    pltpu.sync_copy(rows.at[0], acc_sp.at[didx.at[NCH_W - 1]], add=True)
    plsc.subcore_barrier()
    pltpu.sync_copy(acc_sp.at[pl.ds(s * ROWS_W, ROWS_W)],
                    out_hbm.at[pl.ds(c * NP + s * ROWS_W, ROWS_W)])


def _sel(rows_out, rows_in, odd):
    r = lax.broadcasted_iota(jnp.int32, (rows_out, rows_in), 0)
    c = lax.broadcasted_iota(jnp.int32, (rows_out, rows_in), 1)
    return (c == 2 * r + odd).astype(jnp.float32)


def _selT(rows_out, rows_in, odd):
    r = lax.broadcasted_iota(jnp.int32, (rows_out, rows_in), 0)
    c = lax.broadcasted_iota(jnp.int32, (rows_out, rows_in), 1)
    return (r == 2 * c + odd).astype(jnp.float32)


def _scales(d0_ref, d1_ref):
    i = pl.program_id(0)
    dd = d0_ref[...] + d1_ref[...]
    rn = lax.broadcasted_iota(jnp.int32, (BLK, BLK // 8), 0)
    rc = lax.broadcasted_iota(jnp.int32, (BLK, BLK // 8), 1)
    u = (rc == rn // 8).astype(jnp.float32)
    t = jnp.dot(u, dd, preferred_element_type=jnp.float32)
    cn = lax.broadcasted_iota(jnp.int32, (BLK, 128), 0)
    cc = lax.broadcasted_iota(jnp.int32, (BLK, 128), 1)
    msk = (cc == (cn % 8) * LANES).astype(jnp.float32)
    deg = jnp.sum(t * msk, axis=1, keepdims=True)
    rows = i * BLK + lax.broadcasted_iota(jnp.int32, (BLK, 1), 0)
    ok = (rows < N_NODES) & (deg > 0.0)
    dinv = jnp.where(ok, lax.rsqrt(jnp.maximum(deg, 1e-30)), 0.0)
    lo = jnp.dot(_sel(BLK2, BLK, 0), dinv,
                 preferred_element_type=jnp.float32)
    hi = jnp.dot(_sel(BLK2, BLK, 1), dinv,
                 preferred_element_type=jnp.float32)
    return jnp.concatenate([jnp.broadcast_to(lo, (BLK2, DIM_H)),
                            jnp.broadcast_to(hi, (BLK2, DIM_H))], axis=1)


def _tc1_body(x_ref, w1_ref, m_ref):
    g = jnp.dot(x_ref[...], w1_ref[...], preferred_element_type=jnp.float32)
    lo = jnp.dot(_sel(BLK2, BLK, 0), g, preferred_element_type=jnp.float32)
    hi = jnp.dot(_sel(BLK2, BLK, 1), g, preferred_element_type=jnp.float32)
    m_ref[...] = jnp.concatenate([lo, hi], axis=1)


def _tc1(xp, W1):
    return pl.pallas_call(
        _tc1_body,
        grid=(GRID,),
        in_specs=[
            pl.BlockSpec((BLK, DIM_IN), lambda i: (i, 0)),
            pl.BlockSpec((DIM_IN, DIM_H), lambda i: (0, 0)),
        ],
        out_specs=pl.BlockSpec((BLK2, 128), lambda i: (i, 0)),
        out_shape=jax.ShapeDtypeStruct((NP // 2, 128), jnp.float32),
    )(xp, W1)


def _tc1b_body(m_ref, d0_ref, d1_ref, g_ref):
    g_ref[...] = m_ref[...] * _scales(d0_ref, d1_ref)


def _tc1b(m1p, d0p, d1p):
    return pl.pallas_call(
        _tc1b_body,
        grid=(GRID,),
        in_specs=[
            pl.BlockSpec((BLK2, 128), lambda i: (i, 0)),
            pl.BlockSpec((BLK // 8, 128), lambda i: (i, 0)),
            pl.BlockSpec((BLK // 8, 128), lambda i: (i, 0)),
        ],
        out_specs=pl.BlockSpec((BLK2, 128), lambda i: (i, 0)),
        out_shape=jax.ShapeDtypeStruct((NP // 2, 128), jnp.float32),
    )(m1p, d0p, d1p)


def _tc2_body(p0_ref, p1_ref, d0_ref, d1_ref, b1_ref, w2_ref, g_ref):
    sc2 = _scales(d0_ref, d1_ref)
    a = (p0_ref[...] + p1_ref[...]) * sc2 + b1_ref[...]
    h = jnp.maximum(a, 0.0)
    g_ref[...] = jnp.dot(h, w2_ref[...],
                         preferred_element_type=jnp.float32) * sc2


def _tc2(p0, p1, d0p, d1p, b1p, W2blk):
    return pl.pallas_call(
        _tc2_body,
        grid=(GRID,),
        in_specs=[
            pl.BlockSpec((BLK2, 128), lambda i: (i, 0)),
            pl.BlockSpec((BLK2, 128), lambda i: (i, 0)),
            pl.BlockSpec((BLK // 8, 128), lambda i: (i, 0)),
            pl.BlockSpec((BLK // 8, 128), lambda i: (i, 0)),
            pl.BlockSpec((1, 128), lambda i: (0, 0)),
            pl.BlockSpec((128, 128), lambda i: (0, 0)),
        ],
        out_specs=pl.BlockSpec((BLK2, 128), lambda i: (i, 0)),
        out_shape=jax.ShapeDtypeStruct((NP // 2, 128), jnp.float32),
    )(p0, p1, d0p, d1p, b1p, W2blk)


def _tc3_body(p0_ref, p1_ref, d0_ref, d1_ref, b2_ref, bt_ref, wl_ref, bl_ref,
              fin_ref, acc):
    i = pl.program_id(0)
    sc2 = _scales(d0_ref, d1_ref)
    h2p = jnp.maximum((p0_ref[...] + p1_ref[...]) * sc2 + b2_ref[...], 0.0)
    h2 = (jnp.dot(_selT(BLK, BLK2, 0), h2p[:, :DIM_H],
                  preferred_element_type=jnp.float32) +
          jnp.dot(_selT(BLK, BLK2, 1), h2p[:, DIM_H:],
                  preferred_element_type=jnp.float32))
    bt = bt_ref[0]
    gids = lax.broadcasted_iota(jnp.int32, (N_GRAPH, BLK), 0)
    oh = (bt == gids).astype(jnp.float32)
    haug = jnp.concatenate([h2, jnp.ones((BLK, DIM_H), jnp.float32)], axis=1)
    part = jnp.dot(oh, haug, preferred_element_type=jnp.float32)

    @pl.when(i == 0)
    def _():
        acc[...] = part

    @pl.when(i > 0)
    def _():
        acc[...] += part

    @pl.when(i == GRID - 1)
    def _():
        sums = acc[:, :DIM_H]
        cnt = acc[:, DIM_H:DIM_H + 1]
        pooled = sums / jnp.maximum(cnt, 1.0)
        fin_ref[...] = jnp.dot(pooled, wl_ref[...],
                               preferred_element_type=jnp.float32) + bl_ref[...]


def _tc3(p0, p1, d0p, d1p, b2p, batchp, wlp, blp):
    return pl.pallas_call(
        _tc3_body,
        grid=(GRID,),
        in_specs=[
            pl.BlockSpec((BLK2, 128), lambda i: (i, 0)),
            pl.BlockSpec((BLK2, 128), lambda i: (i, 0)),
            pl.BlockSpec((BLK // 8, 128), lambda i: (i, 0)),
            pl.BlockSpec((BLK // 8, 128), lambda i: (i, 0)),
            pl.BlockSpec((1, 128), lambda i: (0, 0)),
            pl.BlockSpec((1, 1, BLK), lambda i: (i, 0, 0)),
            pl.BlockSpec((DIM_H, 128), lambda i: (0, 0)),
            pl.BlockSpec((1, 128), lambda i: (0, 0)),
        ],
        out_specs=pl.BlockSpec((N_GRAPH, 128), lambda i: (0, 0)),
        out_shape=jax.ShapeDtypeStruct((N_GRAPH, 128), jnp.float32),
        scratch_shapes=[pltpu.VMEM((N_GRAPH, 128), jnp.float32)],
    )(p0, p1, d0p, d1p, b2p, batchp, wlp, blp)


def kernel(x, edge_index, batch, W1, b1, W2, b2, Wlin, blin):
    loop = jnp.arange(N_NODES, dtype=jnp.int32)
    npad = EP - (N_EDGE + N_NODES)
    pad_rows = N_NODES + (jnp.arange(npad, dtype=jnp.int32) % (NP - N_NODES))
    src = jnp.concatenate([edge_index[0], loop, pad_rows]).reshape(NCH_TOT, CH)
    dst = jnp.concatenate([edge_index[1], loop, pad_rows]).reshape(NCH_TOT, CH)

    deg2 = _deg_kernel(dst)
    degp = deg2.reshape(2, NP // 8, 128)
    d0p, d1p = degp[0], degp[1]

    xp = jnp.pad(x, ((0, NP - N_NODES), (0, 0)))
    m1p = _tc1(xp, W1)
    g1 = _tc1b(m1p, d0p, d1p)
    a1 = _agg_kernel(g1.reshape(NP, DIM_H), src, dst)
    a1p = a1.reshape(2, NP // 2, 128)
    b1p = jnp.concatenate([b1, b1]).reshape(1, 128)
    W2blk = jnp.zeros((128, 128), W2.dtype)
    W2blk = W2blk.at[:DIM_H, :DIM_H].set(W2).at[DIM_H:, DIM_H:].set(W2)
    g2 = _tc2(a1p[0], a1p[1], d0p, d1p, b1p, W2blk)
    a2 = _agg_kernel(g2.reshape(NP, DIM_H), src, dst)
    a2p = a2.reshape(2, NP // 2, 128)

    batchp = jnp.pad(batch, (0, NP - N_NODES),
                     constant_values=N_GRAPH).reshape(GRID, 1, BLK)
    b2p = jnp.concatenate([b2, b2]).reshape(1, 128)
    wlp = jnp.pad(Wlin, ((0, 0), (0, 128 - DIM_O)))
    blp = jnp.pad(blin, (0, 128 - DIM_O)).reshape(1, 128)
    fin = _tc3(a2p[0], a2p[1], d0p, d1p, b2p, batchp, wlp, blp)
    return fin[:, :DIM_O]


# speedup vs baseline: 1.2673x; 1.2673x over previous
import functools

import jax
import jax.numpy as jnp
from jax import lax
from jax.experimental import pallas as pl
from jax.experimental.pallas import tpu as pltpu
from jax.experimental.pallas import tpu_sc as plsc

N_NODES = 10000
DIM_IN = 128
DIM_H = 64
DIM_O = 6
N_GRAPH = 64
N_EDGE = 320000

NC, NS, LANES = 2, 16, 16
NW = NC * NS
NP = 10240
ROWS_W = NP // NS
CH = 128
NCH_W = 81
NCH_TOT = NW * NCH_W
EP = NCH_TOT * CH
NB = 4
BLK = 640
BLK2 = BLK // 2
GRID = NP // BLK

_mesh = plsc.VectorSubcoreMesh(
    core_axis_name="c", subcore_axis_name="s", num_cores=NC, num_subcores=NS)
_sc_params = pltpu.CompilerParams(use_tc_tiling_on_sc=False)


@functools.partial(
    pl.kernel,
    out_type=jax.ShapeDtypeStruct((NC * NP, LANES), jnp.float32),
    mesh=_mesh,
    scratch_types=[
        pltpu.VMEM((CH, LANES), jnp.float32),
        pltpu.VMEM((CH, LANES), jnp.float32),
        pltpu.VMEM((NCH_W, CH), jnp.int32),
        pltpu.SemaphoreType.DMA,
        pltpu.VMEM_SHARED((NP, LANES), jnp.float32),
    ],
    compiler_params=_sc_params,
)
def _deg_kernel(dst_hbm, out_hbm, zb, ones_v, didx, ssem, cnt_sp):
    c = lax.axis_index("c")
    s = lax.axis_index("s")
    w = c * NS + s

    def fill(i, _):
        zb[i, :] = jnp.zeros((LANES,), jnp.float32)
        ones_v[i, :] = jnp.ones((LANES,), jnp.float32)
        return 0

    lax.fori_loop(0, CH, fill, 0)
    pltpu.sync_copy(dst_hbm.at[pl.ds(w * NCH_W, NCH_W)], didx)
    for k in range(ROWS_W // CH):
        pltpu.sync_copy(zb, cnt_sp.at[pl.ds(s * ROWS_W + k * CH, CH)])
    plsc.subcore_barrier()

    def pair(i, _):
        for k in range(2):
            j = i * 2 + k
            pltpu.async_copy(ones_v, cnt_sp.at[didx.at[j]], ssem, add=True)

            @pl.when(j >= 4)
            def _():
                pltpu.make_async_copy(ones_v, cnt_sp.at[didx.at[j]],
                                      ssem).wait()
        return 0

    lax.fori_loop(0, (NCH_W - 1) // 2, pair, 0)
    pltpu.async_copy(ones_v, cnt_sp.at[didx.at[NCH_W - 1]], ssem, add=True)
    for _ in range(5):
        pltpu.make_async_copy(ones_v, cnt_sp.at[didx.at[0]], ssem).wait()
    plsc.subcore_barrier()
    pltpu.sync_copy(cnt_sp.at[pl.ds(s * ROWS_W, ROWS_W)],
                    out_hbm.at[pl.ds(c * NP + s * ROWS_W, ROWS_W)])


@functools.partial(
    pl.kernel,
    out_type=jax.ShapeDtypeStruct((NC * NP, DIM_H), jnp.float32),
    mesh=_mesh,
    scratch_types=[
        pltpu.VMEM((CH, DIM_H), jnp.float32),
        pltpu.VMEM((NB, CH, DIM_H), jnp.float32),
        pltpu.VMEM((NCH_W, CH), jnp.int32),
        pltpu.VMEM((NCH_W, CH), jnp.int32),
        pltpu.SemaphoreType.DMA,
        pltpu.VMEM_SHARED((NP, DIM_H), jnp.float32),
    ],
    compiler_params=_sc_params,
)
def _agg_kernel(g_hbm, src_hbm, dst_hbm, out_hbm, zb, rows, sidx, didx, gsem,
                acc_sp):
    c = lax.axis_index("c")
    s = lax.axis_index("s")
    w = c * NS + s

    def fill(i, _):
        for k in range(DIM_H // LANES):
            zb[i, pl.ds(k * LANES, LANES)] = jnp.zeros((LANES,), jnp.float32)
        return 0

    lax.fori_loop(0, CH, fill, 0)
    pltpu.sync_copy(src_hbm.at[pl.ds(w * NCH_W, NCH_W)], sidx)
    pltpu.sync_copy(dst_hbm.at[pl.ds(w * NCH_W, NCH_W)], didx)
    for k in range(ROWS_W // CH):
        pltpu.sync_copy(zb, acc_sp.at[pl.ds(s * ROWS_W + k * CH, CH)])
    plsc.subcore_barrier()

    for b in range(NB - 1):
        pltpu.async_copy(g_hbm.at[sidx.at[b]], rows.at[b], gsem)

    def quad(i, _):
        for k in range(NB):
            j = i * NB + k
            pltpu.make_async_copy(g_hbm.at[sidx.at[k]], rows.at[k],
                                  gsem).wait()
            jn = j + NB - 1

            @pl.when(jn < NCH_W)
            def _():
                bn = (k + NB - 1) % NB
                pltpu.async_copy(g_hbm.at[sidx.at[jn]], rows.at[bn], gsem)

            pltpu.sync_copy(rows.at[k], acc_sp.at[didx.at[j]], add=True)
        return 0

    lax.fori_loop(0, (NCH_W - 1) // NB, quad, 0)
    pltpu.make_async_copy(g_hbm.at[sidx.at[0]], rows.at[0], gsem).wait()
    pltpu.sync_copy(rows.at[0], acc_sp.at[didx.at[NCH_W - 1]], add=True)
    plsc.subcore_barrier()
    pltpu.sync_copy(acc_sp.at[pl.ds(s * ROWS_W, ROWS_W)],
                    out_hbm.at[pl.ds(c * NP + s * ROWS_W, ROWS_W)])


def _sel(rows_out, rows_in, odd):
    r = lax.broadcasted_iota(jnp.int32, (rows_out, rows_in), 0)
    c = lax.broadcasted_iota(jnp.int32, (rows_out, rows_in), 1)
    return (c == 2 * r + odd).astype(jnp.float32)


def _scales(dd):
    i = pl.program_id(0)
    rn = lax.broadcasted_iota(jnp.int32, (BLK, BLK // 8), 0)
    rc = lax.broadcasted_iota(jnp.int32, (BLK, BLK // 8), 1)
    u = (rc == rn // 8).astype(jnp.float32)
    t = jnp.dot(u, dd, preferred_element_type=jnp.float32)
    cn = lax.broadcasted_iota(jnp.int32, (BLK, 128), 0)
    cc = lax.broadcasted_iota(jnp.int32, (BLK, 128), 1)
    msk = (cc == (cn % 8) * LANES).astype(jnp.float32)
    deg = jnp.sum(t * msk, axis=1, keepdims=True)
    rows = i * BLK + lax.broadcasted_iota(jnp.int32, (BLK, 1), 0)
    ok = (rows < N_NODES) & (deg > 0.0)
    dinv = jnp.where(ok, lax.rsqrt(jnp.maximum(deg, 1e-30)), 0.0)
    lo = jnp.dot(_sel(BLK2, BLK, 0), dinv,
                 preferred_element_type=jnp.float32)
    hi = jnp.dot(_sel(BLK2, BLK, 1), dinv,
                 preferred_element_type=jnp.float32)
    return jnp.concatenate([jnp.broadcast_to(lo, (BLK2, DIM_H)),
                            jnp.broadcast_to(hi, (BLK2, DIM_H))], axis=1)


def _tc1_body(x_ref, w1_ref, m_ref):
    g = jnp.dot(x_ref[...], w1_ref[...], preferred_element_type=jnp.float32)
    lo = jnp.dot(_sel(BLK2, BLK, 0), g, preferred_element_type=jnp.float32)
    hi = jnp.dot(_sel(BLK2, BLK, 1), g, preferred_element_type=jnp.float32)
    m_ref[...] = jnp.concatenate([lo, hi], axis=1)


def _tc1(xp, W1):
    return pl.pallas_call(
        _tc1_body,
        grid=(GRID,),
        in_specs=[
            pl.BlockSpec((BLK, DIM_IN), lambda i: (i, 0)),
            pl.BlockSpec((DIM_IN, DIM_H), lambda i: (0, 0)),
        ],
        out_specs=pl.BlockSpec((BLK2, 128), lambda i: (i, 0)),
        out_shape=jax.ShapeDtypeStruct((NP // 2, 128), jnp.float32),
    )(xp, W1)


def _tcd_body(m_ref, deg_ref, sc_ref, g_ref):
    dd = deg_ref[0] + deg_ref[1]
    sc2 = _scales(dd)
    sc_ref[...] = sc2
    g_ref[...] = m_ref[...] * sc2


def _tcd(m1p, deg2):
    return pl.pallas_call(
        _tcd_body,
        grid=(GRID,),
        in_specs=[
            pl.BlockSpec((BLK2, 128), lambda i: (i, 0)),
            pl.BlockSpec((NC, BLK // 8, 128), lambda i: (0, i, 0)),
        ],
        out_specs=[
            pl.BlockSpec((BLK2, 128), lambda i: (i, 0)),
            pl.BlockSpec((BLK2, 128), lambda i: (i, 0)),
        ],
        out_shape=[
            jax.ShapeDtypeStruct((NP // 2, 128), jnp.float32),
            jax.ShapeDtypeStruct((NP // 2, 128), jnp.float32),
        ],
    )(m1p, deg2)


def _tc2_body(a_ref, sc_ref, b1_ref, w2_ref, g_ref):
    sc2 = sc_ref[...]
    a = (a_ref[0] + a_ref[1]) * sc2 + b1_ref[...]
    h = jnp.maximum(a, 0.0)
    g_ref[...] = jnp.dot(h, w2_ref[...],
                         preferred_element_type=jnp.float32) * sc2


def _tc2(a1, scp, b1p, W2blk):
    return pl.pallas_call(
        _tc2_body,
        grid=(GRID,),
        in_specs=[
            pl.BlockSpec((NC, BLK2, 128), lambda i: (0, i, 0)),
            pl.BlockSpec((BLK2, 128), lambda i: (i, 0)),
            pl.BlockSpec((1, 128), lambda i: (0, 0)),
            pl.BlockSpec((128, 128), lambda i: (0, 0)),
        ],
        out_specs=pl.BlockSpec((BLK2, 128), lambda i: (i, 0)),
        out_shape=jax.ShapeDtypeStruct((NP // 2, 128), jnp.float32),
    )(a1, scp, b1p, W2blk)


def _tc3_body(a_ref, sc_ref, b2_ref, bt_ref, wl_ref, bl_ref, fin_ref, acc):
    i = pl.program_id(0)
    sc2 = sc_ref[...]
    h2p = jnp.maximum((a_ref[0] + a_ref[1]) * sc2 + b2_ref[...], 0.0)
    h2cat = jnp.concatenate([h2p[:, :DIM_H], h2p[:, DIM_H:]], axis=0)
    bt = bt_ref[0]
    gids = lax.broadcasted_iota(jnp.int32, (N_GRAPH, BLK), 0)
    oh = (bt == gids).astype(jnp.float32)
    haug = jnp.concatenate([h2cat, jnp.ones((BLK, DIM_H), jnp.float32)],
                           axis=1)
    part = jnp.dot(oh, haug, preferred_element_type=jnp.float32)

    @pl.when(i == 0)
    def _():
        acc[...] = part

    @pl.when(i > 0)
    def _():
        acc[...] += part

    @pl.when(i == GRID - 1)
    def _():
        sums = acc[:, :DIM_H]
        cnt = acc[:, DIM_H:DIM_H + 1]
        pooled = sums / jnp.maximum(cnt, 1.0)
        fin_ref[...] = jnp.dot(pooled, wl_ref[...],
                               preferred_element_type=jnp.float32) + bl_ref[...]


def _tc3(a2, scp, b2p, batchp, wlp, blp):
    return pl.pallas_call(
        _tc3_body,
        grid=(GRID,),
        in_specs=[
            pl.BlockSpec((NC, BLK2, 128), lambda i: (0, i, 0)),
            pl.BlockSpec((BLK2, 128), lambda i: (i, 0)),
            pl.BlockSpec((1, 128), lambda i: (0, 0)),
            pl.BlockSpec((1, 1, BLK), lambda i: (i, 0, 0)),
            pl.BlockSpec((DIM_H, 128), lambda i: (0, 0)),
            pl.BlockSpec((1, 128), lambda i: (0, 0)),
        ],
        out_specs=pl.BlockSpec((N_GRAPH, 128), lambda i: (0, 0)),
        out_shape=jax.ShapeDtypeStruct((N_GRAPH, 128), jnp.float32),
        scratch_shapes=[pltpu.VMEM((N_GRAPH, 128), jnp.float32)],
    )(a2, scp, b2p, batchp, wlp, blp)


def kernel(x, edge_index, batch, W1, b1, W2, b2, Wlin, blin):
    loop = jnp.arange(N_NODES, dtype=jnp.int32)
    npad = EP - (N_EDGE + N_NODES)
    pad_rows = N_NODES + (jnp.arange(npad, dtype=jnp.int32) % (NP - N_NODES))
    src = jnp.concatenate([edge_index[0], loop, pad_rows]).reshape(NCH_TOT, CH)
    dst = jnp.concatenate([edge_index[1], loop, pad_rows]).reshape(NCH_TOT, CH)

    deg2 = _deg_kernel(dst).reshape(NC, NP // 8, 128)

    xp = jnp.pad(x, ((0, NP - N_NODES), (0, 0)))
    m1p = _tc1(xp, W1)
    scp, g1 = _tcd(m1p, deg2)
    a1 = _agg_kernel(g1.reshape(NP, DIM_H), src, dst).reshape(NC, NP // 2, 128)
    b1p = jnp.concatenate([b1, b1]).reshape(1, 128)
    W2blk = jnp.zeros((128, 128), W2.dtype)
    W2blk = W2blk.at[:DIM_H, :DIM_H].set(W2).at[DIM_H:, DIM_H:].set(W2)
    g2 = _tc2(a1, scp, b1p, W2blk)
    a2 = _agg_kernel(g2.reshape(NP, DIM_H), src, dst).reshape(NC, NP // 2, 128)

    batchp = jnp.pad(batch, (0, NP - N_NODES), constant_values=N_GRAPH
                     ).reshape(GRID, BLK2, 2).transpose(0, 2, 1
                     ).reshape(GRID, 1, BLK)
    b2p = jnp.concatenate([b2, b2]).reshape(1, 128)
    wlp = jnp.pad(Wlin, ((0, 0), (0, 128 - DIM_O)))
    blp = jnp.pad(blin, (0, 128 - DIM_O)).reshape(1, 128)
    fin = _tc3(a2, scp, b2p, batchp, wlp, blp)
    return fin[:, :DIM_O]
